# baseline (speedup 1.0000x reference)
import functools

import jax
import jax.numpy as jnp
from jax import lax
from jax.experimental import pallas as pl
from jax.experimental.pallas import tpu as pltpu
from jax.experimental.pallas import tpu_sc as plsc

N = 10000
D = 128
R = 10240
NC = 2
NS = 16
NW = NC * NS
CH = 128
NCHUNK = 80
E_PAD = NW * NCHUNK * CH
TOT_CHUNK = E_PAD // CH
CPW0 = 96
CPW1 = 2 * NCHUNK - CPW0
ROWS_PER_TILE = R // NS

_MESH = plsc.VectorSubcoreMesh(
    core_axis_name="c", subcore_axis_name="s", num_cores=NC, num_subcores=NS)


_PH = 32


def _chunk_start(c, s):
  return jnp.where(c == 0, s * CPW0, NS * CPW0 + s * CPW1)


def _phase_sizes(cnt):
  sizes = [_PH] * (cnt // _PH)
  if cnt % _PH:
    sizes.append(cnt % _PH)
  return sizes


@functools.partial(
    pl.kernel,
    out_type=(
        jax.ShapeDtypeStruct((NW * R,), jnp.float32),
        jax.ShapeDtypeStruct((NW * R,), jnp.float32),
    ),
    mesh=_MESH,
    scratch_types=[
        pltpu.VMEM((_PH, CH), jnp.int32),
        pltpu.VMEM((_PH, CH), jnp.int32),
        pltpu.VMEM((R,), jnp.float32),
        pltpu.VMEM((R,), jnp.float32),
    ],
    compiler_params=pltpu.CompilerParams(needs_layout_passes=False),
)
def _deg_kernel(src_hbm, dst_hbm, od_out, id_out,
                src_v, dst_v, od_h, id_h):
  c = lax.axis_index("c")
  s = lax.axis_index("s")
  w = c * NS + s
  start = _chunk_start(c, s)

  zero = jnp.zeros((16,), jnp.float32)

  def zbody(i, carry):
    od_h[pl.ds(i * 16, 16)] = zero
    id_h[pl.ds(i * 16, 16)] = zero
    return carry

  lax.fori_loop(0, R // 16, zbody, 0)

  ones = jnp.ones((16,), jnp.float32)

  def hist_phases(cnt):
    done = 0
    for sz in _phase_sizes(cnt):
      base = start + done
      pltpu.sync_copy(src_hbm.at[pl.ds(base, sz)], src_v.at[pl.ds(0, sz)])
      pltpu.sync_copy(dst_hbm.at[pl.ds(base, sz)], dst_v.at[pl.ds(0, sz)])

      def ebody(j, carry):
        for k in range(CH // 16):
          sidx = src_v[j, pl.ds(k * 16, 16)]
          plsc.addupdate_scatter(od_h, [sidx], ones)
          didx = dst_v[j, pl.ds(k * 16, 16)]
          plsc.addupdate_scatter(id_h, [didx], ones)
        return carry

      lax.fori_loop(0, sz, ebody, 0)
      done += sz

  @pl.when(c == 0)
  def _():
    hist_phases(CPW0)

  @pl.when(c == 1)
  def _():
    hist_phases(CPW1)

  pltpu.sync_copy(od_h, od_out.at[pl.ds(w * R, R)])
  pltpu.sync_copy(id_h, id_out.at[pl.ds(w * R, R)])


@functools.partial(
    pl.kernel,
    out_type=jax.ShapeDtypeStruct((NC, R, D), jnp.float32),
    mesh=_MESH,
    scratch_types=[
        pltpu.VMEM((_PH, CH), jnp.int32),
        pltpu.VMEM((_PH, CH), jnp.int32),
        pltpu.VMEM((CH, D), jnp.float32),
        pltpu.VMEM((CH, D), jnp.float32),
        pltpu.VMEM_SHARED((R, D), jnp.float32),
        pltpu.SemaphoreType.DMA,
        pltpu.SemaphoreType.DMA,
    ],
)
def _spmm_kernel(x_hbm, src_hbm, dst_hbm, zrow_hbm,
                 out_hbm,
                 src_v, dst_v, rows0_v, rows1_v, acc_s, sem0, sem1):
  c = lax.axis_index("c")
  s = lax.axis_index("s")
  row0 = s * ROWS_PER_TILE
  start = _chunk_start(c, s)
  pltpu.sync_copy(zrow_hbm, acc_s.at[pl.ds(row0, ROWS_PER_TILE)])
  plsc.subcore_barrier()

  def edge_phases(cnt):
    done = 0
    for sz in _phase_sizes(cnt):
      base = start + done
      pltpu.sync_copy(src_hbm.at[pl.ds(base, sz)], src_v.at[pl.ds(0, sz)])
      pltpu.sync_copy(dst_hbm.at[pl.ds(base, sz)], dst_v.at[pl.ds(0, sz)])
      pltpu.async_copy(x_hbm.at[src_v.at[0]], rows0_v, sem0)
      pltpu.async_copy(x_hbm.at[src_v.at[1]], rows1_v, sem1)

      def body(i, carry):
        j0 = 2 * i
        pltpu.make_async_copy(x_hbm.at[src_v.at[j0]], rows0_v, sem0).wait()
        pltpu.sync_copy(rows0_v, acc_s.at[dst_v.at[j0]], add=True)

        @pl.when(j0 + 2 < sz)
        def _():
          pltpu.async_copy(x_hbm.at[src_v.at[j0 + 2]], rows0_v, sem0)

        pltpu.make_async_copy(x_hbm.at[src_v.at[j0 + 1]], rows1_v, sem1).wait()
        pltpu.sync_copy(rows1_v, acc_s.at[dst_v.at[j0 + 1]], add=True)

        @pl.when(j0 + 3 < sz)
        def _():
          pltpu.async_copy(x_hbm.at[src_v.at[j0 + 3]], rows1_v, sem1)

        return carry

      lax.fori_loop(0, sz // 2, body, 0)
      done += sz

  @pl.when(c == 0)
  def _():
    edge_phases(CPW0)

  @pl.when(c == 1)
  def _():
    edge_phases(CPW1)
  plsc.subcore_barrier()
  sl = pl.ds(row0, ROWS_PER_TILE)
  pltpu.sync_copy(acc_s.at[sl], out_hbm.at[c, sl])


_BLK = 2048
_NBLK = R // _BLK


def _colsum(ref):
  s = jnp.sum(ref[...], axis=0, keepdims=True)
  return lax.transpose(s, (1, 0))


def _layer1_body(h_ref, od_ref, mask_ref, w_ref, out_ref):
  od = _colsum(od_ref)
  ns = lax.rsqrt(jnp.maximum(od, 1.0)) * mask_ref[...]
  x = h_ref[...] * ns
  out_ref[...] = lax.dot_general(
      x, w_ref[...], (((1,), (0,)), ((), ())),
      preferred_element_type=jnp.float32)


def _tc_layer1(h_pad, od2d, mask, W1):
  return pl.pallas_call(
      _layer1_body,
      grid=(_NBLK,),
      in_specs=[
          pl.BlockSpec((_BLK, D), lambda i: (i, 0)),
          pl.BlockSpec((NW, _BLK), lambda i: (0, i)),
          pl.BlockSpec((_BLK, 1), lambda i: (i, 0)),
          pl.BlockSpec((D, D), lambda i: (0, 0)),
      ],
      out_specs=pl.BlockSpec((_BLK, D), lambda i: (i, 0)),
      out_shape=jax.ShapeDtypeStruct((R, D), jnp.float32),
  )(h_pad, od2d, mask, W1)


def _layer2_body(agg_ref, od_ref, id_ref, mask_ref, b_ref, w_ref, out_ref):
  a = agg_ref[0] + agg_ref[1]
  nd = lax.rsqrt(jnp.maximum(_colsum(id_ref), 1.0))
  y = jnp.maximum(a * nd + b_ref[...], 0.0)
  ns = lax.rsqrt(jnp.maximum(_colsum(od_ref), 1.0)) * mask_ref[...]
  out_ref[...] = lax.dot_general(
      y * ns, w_ref[...], (((1,), (0,)), ((), ())),
      preferred_element_type=jnp.float32)


def _tc_layer2(agg1, od2d, id2d, mask, b1, W2):
  return pl.pallas_call(
      _layer2_body,
      grid=(_NBLK,),
      in_specs=[
          pl.BlockSpec((NC, _BLK, D), lambda i: (0, i, 0)),
          pl.BlockSpec((NW, _BLK), lambda i: (0, i)),
          pl.BlockSpec((NW, _BLK), lambda i: (0, i)),
          pl.BlockSpec((_BLK, 1), lambda i: (i, 0)),
          pl.BlockSpec((1, D), lambda i: (0, 0)),
          pl.BlockSpec((D, D), lambda i: (0, 0)),
      ],
      out_specs=pl.BlockSpec((_BLK, D), lambda i: (i, 0)),
      out_shape=jax.ShapeDtypeStruct((R, D), jnp.float32),
  )(agg1, od2d, id2d, mask, b1, W2)


def _head_body(agg_ref, id_ref, mask_ref, b_ref, wfc_ref, bfc_ref, out_ref,
               acc_ref):
  i = pl.program_id(0)

  @pl.when(i == 0)
  def _():
    acc_ref[...] = jnp.zeros_like(acc_ref)

  a = agg_ref[0] + agg_ref[1]
  nd = lax.rsqrt(jnp.maximum(_colsum(id_ref), 1.0))
  y = jnp.maximum(a * nd + b_ref[...], 0.0) * mask_ref[...]
  acc_ref[...] += jnp.sum(y, axis=0, keepdims=True)

  @pl.when(i == _NBLK - 1)
  def _():
    pooled = acc_ref[...] * (1.0 / N)
    out_ref[...] = jnp.sum(pooled * wfc_ref[...], axis=1, keepdims=True) \
        + bfc_ref[...]


def _tc_head(agg2, id2d, mask, b2, wfcT, bfc):
  return pl.pallas_call(
      _head_body,
      grid=(_NBLK,),
      in_specs=[
          pl.BlockSpec((NC, _BLK, D), lambda i: (0, i, 0)),
          pl.BlockSpec((NW, _BLK), lambda i: (0, i)),
          pl.BlockSpec((_BLK, 1), lambda i: (i, 0)),
          pl.BlockSpec((1, D), lambda i: (0, 0)),
          pl.BlockSpec((1, D), lambda i: (0, 0)),
          pl.BlockSpec((1, 1), lambda i: (0, 0)),
      ],
      out_specs=pl.BlockSpec((1, 1), lambda i: (0, 0)),
      out_shape=jax.ShapeDtypeStruct((1, 1), jnp.float32),
      scratch_shapes=[pltpu.VMEM((1, D), jnp.float32)],
  )(agg2, id2d, mask, b2, wfcT, bfc)


def kernel(h, edge_index, W1, b1, W2, b2, Wfc, bfc):
  ei = edge_index.astype(jnp.int32)
  pad = jnp.full((E_PAD - ei.shape[1],), N, dtype=jnp.int32)
  src = jnp.concatenate([ei[0], pad]).reshape(TOT_CHUNK, CH)
  dst = jnp.concatenate([ei[1], pad]).reshape(TOT_CHUNK, CH)
  h_pad = jnp.pad(h, ((0, R - N), (0, 0)))
  mask = jnp.pad(jnp.ones((N, 1), jnp.float32), ((0, R - N), (0, 0)))
  zrow = jnp.zeros((ROWS_PER_TILE, D), jnp.float32)

  od1, id1 = _deg_kernel(src, dst)
  od2d = od1.reshape(NW, R)
  id2d = id1.reshape(NW, R)

  xw1 = _tc_layer1(h_pad, od2d, mask, W1)
  agg1 = _spmm_kernel(xw1, src, dst, zrow)
  xw2 = _tc_layer2(agg1, od2d, id2d, mask, b1.reshape(1, D), W2)
  agg2 = _spmm_kernel(xw2, src, dst, zrow)
  out = _tc_head(agg2, id2d, mask, b2.reshape(1, D), Wfc.reshape(1, D),
                 bfc.reshape(1, 1))
  return out.reshape(1)


# speedup vs baseline: 1.1033x; 1.1033x over previous
import functools

import jax
import jax.numpy as jnp
from jax import lax
from jax.experimental import pallas as pl
from jax.experimental.pallas import tpu as pltpu
from jax.experimental.pallas import tpu_sc as plsc

N = 10000
D = 128
R = 10240
NC = 2
NS = 16
NW = NC * NS
CH = 128
NCHUNK = 80
E_PAD = NW * NCHUNK * CH
TOT_CHUNK = E_PAD // CH
CPW0 = 144
CPW1 = 2 * NCHUNK - CPW0
ROWS_PER_TILE = R // NS

_MESH = plsc.VectorSubcoreMesh(
    core_axis_name="c", subcore_axis_name="s", num_cores=NC, num_subcores=NS)


_PH = 32


def _chunk_start(c, s):
  return jnp.where(c == 0, s * CPW0, NS * CPW0 + s * CPW1)


def _phase_sizes(cnt):
  sizes = [_PH] * (cnt // _PH)
  if cnt % _PH:
    sizes.append(cnt % _PH)
  return sizes


@functools.partial(
    pl.kernel,
    out_type=(
        jax.ShapeDtypeStruct((NW * R,), jnp.float32),
        jax.ShapeDtypeStruct((NW * R,), jnp.float32),
    ),
    mesh=_MESH,
    scratch_types=[
        pltpu.VMEM((_PH, CH), jnp.int32),
        pltpu.VMEM((_PH, CH), jnp.int32),
        pltpu.VMEM((R,), jnp.float32),
        pltpu.VMEM((R,), jnp.float32),
    ],
    compiler_params=pltpu.CompilerParams(needs_layout_passes=False),
)
def _deg_kernel(src_hbm, dst_hbm, od_out, id_out,
                src_v, dst_v, od_h, id_h):
  c = lax.axis_index("c")
  s = lax.axis_index("s")
  w = c * NS + s
  start = _chunk_start(c, s)

  zero = jnp.zeros((16,), jnp.float32)

  def zbody(i, carry):
    od_h[pl.ds(i * 16, 16)] = zero
    id_h[pl.ds(i * 16, 16)] = zero
    return carry

  lax.fori_loop(0, R // 16, zbody, 0)

  ones = jnp.ones((16,), jnp.float32)

  def hist_phases(cnt):
    done = 0
    for sz in _phase_sizes(cnt):
      base = start + done
      pltpu.sync_copy(src_hbm.at[pl.ds(base, sz)], src_v.at[pl.ds(0, sz)])
      pltpu.sync_copy(dst_hbm.at[pl.ds(base, sz)], dst_v.at[pl.ds(0, sz)])

      def ebody(j, carry):
        for k in range(CH // 16):
          sidx = src_v[j, pl.ds(k * 16, 16)]
          plsc.addupdate_scatter(od_h, [sidx], ones)
          didx = dst_v[j, pl.ds(k * 16, 16)]
          plsc.addupdate_scatter(id_h, [didx], ones)
        return carry

      lax.fori_loop(0, sz, ebody, 0)
      done += sz

  @pl.when(c == 0)
  def _():
    hist_phases(CPW0)

  @pl.when(c == 1)
  def _():
    hist_phases(CPW1)

  pltpu.sync_copy(od_h, od_out.at[pl.ds(w * R, R)])
  pltpu.sync_copy(id_h, id_out.at[pl.ds(w * R, R)])


@functools.partial(
    pl.kernel,
    out_type=jax.ShapeDtypeStruct((NC, R, D), jnp.float32),
    mesh=_MESH,
    scratch_types=[
        pltpu.VMEM((_PH, CH), jnp.int32),
        pltpu.VMEM((_PH, CH), jnp.int32),
        pltpu.VMEM((CH, D), jnp.float32),
        pltpu.VMEM((CH, D), jnp.float32),
        pltpu.VMEM_SHARED((R, D), jnp.float32),
        pltpu.SemaphoreType.DMA,
        pltpu.SemaphoreType.DMA,
    ],
)
def _spmm_kernel(x_hbm, src_hbm, dst_hbm, zrow_hbm,
                 out_hbm,
                 src_v, dst_v, rows0_v, rows1_v, acc_s, sem0, sem1):
  c = lax.axis_index("c")
  s = lax.axis_index("s")
  row0 = s * ROWS_PER_TILE
  start = _chunk_start(c, s)
  pltpu.sync_copy(zrow_hbm, acc_s.at[pl.ds(row0, ROWS_PER_TILE)])
  plsc.subcore_barrier()

  def edge_phases(cnt):
    done = 0
    for sz in _phase_sizes(cnt):
      base = start + done
      pltpu.sync_copy(src_hbm.at[pl.ds(base, sz)], src_v.at[pl.ds(0, sz)])
      pltpu.sync_copy(dst_hbm.at[pl.ds(base, sz)], dst_v.at[pl.ds(0, sz)])
      pltpu.async_copy(x_hbm.at[src_v.at[0]], rows0_v, sem0)
      pltpu.async_copy(x_hbm.at[src_v.at[1]], rows1_v, sem1)

      def body(i, carry):
        j0 = 2 * i
        pltpu.make_async_copy(x_hbm.at[src_v.at[j0]], rows0_v, sem0).wait()
        pltpu.sync_copy(rows0_v, acc_s.at[dst_v.at[j0]], add=True)

        @pl.when(j0 + 2 < sz)
        def _():
          pltpu.async_copy(x_hbm.at[src_v.at[j0 + 2]], rows0_v, sem0)

        pltpu.make_async_copy(x_hbm.at[src_v.at[j0 + 1]], rows1_v, sem1).wait()
        pltpu.sync_copy(rows1_v, acc_s.at[dst_v.at[j0 + 1]], add=True)

        @pl.when(j0 + 3 < sz)
        def _():
          pltpu.async_copy(x_hbm.at[src_v.at[j0 + 3]], rows1_v, sem1)

        return carry

      lax.fori_loop(0, sz // 2, body, 0)
      done += sz

  @pl.when(c == 0)
  def _():
    edge_phases(CPW0)

  @pl.when(c == 1)
  def _():
    edge_phases(CPW1)
  plsc.subcore_barrier()
  sl = pl.ds(row0, ROWS_PER_TILE)
  pltpu.sync_copy(acc_s.at[sl], out_hbm.at[c, sl])


_BLK = 2048
_NBLK = R // _BLK


def _colsum(ref):
  s = jnp.sum(ref[...], axis=0, keepdims=True)
  return lax.transpose(s, (1, 0))


def _layer1_body(h_ref, od_ref, mask_ref, w_ref, out_ref):
  od = _colsum(od_ref)
  ns = lax.rsqrt(jnp.maximum(od, 1.0)) * mask_ref[...]
  x = h_ref[...] * ns
  out_ref[...] = lax.dot_general(
      x, w_ref[...], (((1,), (0,)), ((), ())),
      preferred_element_type=jnp.float32)


def _tc_layer1(h_pad, od2d, mask, W1):
  return pl.pallas_call(
      _layer1_body,
      grid=(_NBLK,),
      in_specs=[
          pl.BlockSpec((_BLK, D), lambda i: (i, 0)),
          pl.BlockSpec((NW, _BLK), lambda i: (0, i)),
          pl.BlockSpec((_BLK, 1), lambda i: (i, 0)),
          pl.BlockSpec((D, D), lambda i: (0, 0)),
      ],
      out_specs=pl.BlockSpec((_BLK, D), lambda i: (i, 0)),
      out_shape=jax.ShapeDtypeStruct((R, D), jnp.float32),
  )(h_pad, od2d, mask, W1)


def _layer2_body(agg_ref, od_ref, id_ref, mask_ref, b_ref, w_ref, out_ref):
  a = agg_ref[0] + agg_ref[1]
  nd = lax.rsqrt(jnp.maximum(_colsum(id_ref), 1.0))
  y = jnp.maximum(a * nd + b_ref[...], 0.0)
  ns = lax.rsqrt(jnp.maximum(_colsum(od_ref), 1.0)) * mask_ref[...]
  out_ref[...] = lax.dot_general(
      y * ns, w_ref[...], (((1,), (0,)), ((), ())),
      preferred_element_type=jnp.float32)


def _tc_layer2(agg1, od2d, id2d, mask, b1, W2):
  return pl.pallas_call(
      _layer2_body,
      grid=(_NBLK,),
      in_specs=[
          pl.BlockSpec((NC, _BLK, D), lambda i: (0, i, 0)),
          pl.BlockSpec((NW, _BLK), lambda i: (0, i)),
          pl.BlockSpec((NW, _BLK), lambda i: (0, i)),
          pl.BlockSpec((_BLK, 1), lambda i: (i, 0)),
          pl.BlockSpec((1, D), lambda i: (0, 0)),
          pl.BlockSpec((D, D), lambda i: (0, 0)),
      ],
      out_specs=pl.BlockSpec((_BLK, D), lambda i: (i, 0)),
      out_shape=jax.ShapeDtypeStruct((R, D), jnp.float32),
  )(agg1, od2d, id2d, mask, b1, W2)


def _head_body(agg_ref, id_ref, mask_ref, b_ref, wfc_ref, bfc_ref, out_ref,
               acc_ref):
  i = pl.program_id(0)

  @pl.when(i == 0)
  def _():
    acc_ref[...] = jnp.zeros_like(acc_ref)

  a = agg_ref[0] + agg_ref[1]
  nd = lax.rsqrt(jnp.maximum(_colsum(id_ref), 1.0))
  y = jnp.maximum(a * nd + b_ref[...], 0.0) * mask_ref[...]
  acc_ref[...] += jnp.sum(y, axis=0, keepdims=True)

  @pl.when(i == _NBLK - 1)
  def _():
    pooled = acc_ref[...] * (1.0 / N)
    out_ref[...] = jnp.sum(pooled * wfc_ref[...], axis=1, keepdims=True) \
        + bfc_ref[...]


def _tc_head(agg2, id2d, mask, b2, wfcT, bfc):
  return pl.pallas_call(
      _head_body,
      grid=(_NBLK,),
      in_specs=[
          pl.BlockSpec((NC, _BLK, D), lambda i: (0, i, 0)),
          pl.BlockSpec((NW, _BLK), lambda i: (0, i)),
          pl.BlockSpec((_BLK, 1), lambda i: (i, 0)),
          pl.BlockSpec((1, D), lambda i: (0, 0)),
          pl.BlockSpec((1, D), lambda i: (0, 0)),
          pl.BlockSpec((1, 1), lambda i: (0, 0)),
      ],
      out_specs=pl.BlockSpec((1, 1), lambda i: (0, 0)),
      out_shape=jax.ShapeDtypeStruct((1, 1), jnp.float32),
      scratch_shapes=[pltpu.VMEM((1, D), jnp.float32)],
  )(agg2, id2d, mask, b2, wfcT, bfc)


def kernel(h, edge_index, W1, b1, W2, b2, Wfc, bfc):
  ei = edge_index.astype(jnp.int32)
  pad = jnp.full((E_PAD - ei.shape[1],), N, dtype=jnp.int32)
  src = jnp.concatenate([ei[0], pad]).reshape(TOT_CHUNK, CH)
  dst = jnp.concatenate([ei[1], pad]).reshape(TOT_CHUNK, CH)
  h_pad = jnp.pad(h, ((0, R - N), (0, 0)))
  mask = jnp.pad(jnp.ones((N, 1), jnp.float32), ((0, R - N), (0, 0)))
  zrow = jnp.zeros((ROWS_PER_TILE, D), jnp.float32)

  od1, id1 = _deg_kernel(src, dst)
  od2d = od1.reshape(NW, R)
  id2d = id1.reshape(NW, R)

  xw1 = _tc_layer1(h_pad, od2d, mask, W1)
  agg1 = _spmm_kernel(xw1, src, dst, zrow)
  xw2 = _tc_layer2(agg1, od2d, id2d, mask, b1.reshape(1, D), W2)
  agg2 = _spmm_kernel(xw2, src, dst, zrow)
  out = _tc_head(agg2, id2d, mask, b2.reshape(1, D), Wfc.reshape(1, D),
                 bfc.reshape(1, 1))
  return out.reshape(1)


# speedup vs baseline: 1.1082x; 1.0045x over previous
import functools

import jax
import jax.numpy as jnp
from jax import lax
from jax.experimental import pallas as pl
from jax.experimental.pallas import tpu as pltpu
from jax.experimental.pallas import tpu_sc as plsc

N = 10000
D = 128
R = 10240
NC = 2
NS = 16
NW = NC * NS
CH = 128
NCHUNK = 80
E_PAD = NW * NCHUNK * CH
TOT_CHUNK = E_PAD // CH
CPW0 = 152
CPW1 = 2 * NCHUNK - CPW0
ROWS_PER_TILE = R // NS

_MESH = plsc.VectorSubcoreMesh(
    core_axis_name="c", subcore_axis_name="s", num_cores=NC, num_subcores=NS)


_PH = 32


def _chunk_start(c, s):
  return jnp.where(c == 0, s * CPW0, NS * CPW0 + s * CPW1)


def _phase_sizes(cnt):
  sizes = [_PH] * (cnt // _PH)
  if cnt % _PH:
    sizes.append(cnt % _PH)
  return sizes


@functools.partial(
    pl.kernel,
    out_type=(
        jax.ShapeDtypeStruct((NW * R,), jnp.float32),
        jax.ShapeDtypeStruct((NW * R,), jnp.float32),
    ),
    mesh=_MESH,
    scratch_types=[
        pltpu.VMEM((_PH, CH), jnp.int32),
        pltpu.VMEM((_PH, CH), jnp.int32),
        pltpu.VMEM((R,), jnp.float32),
        pltpu.VMEM((R,), jnp.float32),
    ],
    compiler_params=pltpu.CompilerParams(needs_layout_passes=False),
)
def _deg_kernel(src_hbm, dst_hbm, od_out, id_out,
                src_v, dst_v, od_h, id_h):
  c = lax.axis_index("c")
  s = lax.axis_index("s")
  w = c * NS + s
  start = _chunk_start(c, s)

  zero = jnp.zeros((16,), jnp.float32)

  def zbody(i, carry):
    od_h[pl.ds(i * 16, 16)] = zero
    id_h[pl.ds(i * 16, 16)] = zero
    return carry

  lax.fori_loop(0, R // 16, zbody, 0)

  ones = jnp.ones((16,), jnp.float32)

  def hist_phases(cnt):
    done = 0
    for sz in _phase_sizes(cnt):
      base = start + done
      pltpu.sync_copy(src_hbm.at[pl.ds(base, sz)], src_v.at[pl.ds(0, sz)])
      pltpu.sync_copy(dst_hbm.at[pl.ds(base, sz)], dst_v.at[pl.ds(0, sz)])

      def ebody(j, carry):
        for k in range(CH // 16):
          sidx = src_v[j, pl.ds(k * 16, 16)]
          plsc.addupdate_scatter(od_h, [sidx], ones)
          didx = dst_v[j, pl.ds(k * 16, 16)]
          plsc.addupdate_scatter(id_h, [didx], ones)
        return carry

      lax.fori_loop(0, sz, ebody, 0)
      done += sz

  @pl.when(c == 0)
  def _():
    hist_phases(CPW0)

  @pl.when(c == 1)
  def _():
    hist_phases(CPW1)

  pltpu.sync_copy(od_h, od_out.at[pl.ds(w * R, R)])
  pltpu.sync_copy(id_h, id_out.at[pl.ds(w * R, R)])


@functools.partial(
    pl.kernel,
    out_type=jax.ShapeDtypeStruct((NC, R, D), jnp.float32),
    mesh=_MESH,
    scratch_types=[
        pltpu.VMEM((_PH, CH), jnp.int32),
        pltpu.VMEM((_PH, CH), jnp.int32),
        pltpu.VMEM((CH, D), jnp.float32),
        pltpu.VMEM((CH, D), jnp.float32),
        pltpu.VMEM_SHARED((R, D), jnp.float32),
        pltpu.SemaphoreType.DMA,
        pltpu.SemaphoreType.DMA,
    ],
)
def _spmm_kernel(x_hbm, src_hbm, dst_hbm, zrow_hbm,
                 out_hbm,
                 src_v, dst_v, rows0_v, rows1_v, acc_s, sem0, sem1):
  c = lax.axis_index("c")
  s = lax.axis_index("s")
  row0 = s * ROWS_PER_TILE
  start = _chunk_start(c, s)
  pltpu.sync_copy(zrow_hbm, acc_s.at[pl.ds(row0, ROWS_PER_TILE)])
  plsc.subcore_barrier()

  def edge_phases(cnt):
    done = 0
    for sz in _phase_sizes(cnt):
      base = start + done
      pltpu.sync_copy(src_hbm.at[pl.ds(base, sz)], src_v.at[pl.ds(0, sz)])
      pltpu.sync_copy(dst_hbm.at[pl.ds(base, sz)], dst_v.at[pl.ds(0, sz)])
      pltpu.async_copy(x_hbm.at[src_v.at[0]], rows0_v, sem0)
      pltpu.async_copy(x_hbm.at[src_v.at[1]], rows1_v, sem1)

      def body(i, carry):
        j0 = 2 * i
        pltpu.make_async_copy(x_hbm.at[src_v.at[j0]], rows0_v, sem0).wait()
        pltpu.sync_copy(rows0_v, acc_s.at[dst_v.at[j0]], add=True)

        @pl.when(j0 + 2 < sz)
        def _():
          pltpu.async_copy(x_hbm.at[src_v.at[j0 + 2]], rows0_v, sem0)

        pltpu.make_async_copy(x_hbm.at[src_v.at[j0 + 1]], rows1_v, sem1).wait()
        pltpu.sync_copy(rows1_v, acc_s.at[dst_v.at[j0 + 1]], add=True)

        @pl.when(j0 + 3 < sz)
        def _():
          pltpu.async_copy(x_hbm.at[src_v.at[j0 + 3]], rows1_v, sem1)

        return carry

      lax.fori_loop(0, sz // 2, body, 0)
      done += sz

  @pl.when(c == 0)
  def _():
    edge_phases(CPW0)

  @pl.when(c == 1)
  def _():
    edge_phases(CPW1)
  plsc.subcore_barrier()
  sl = pl.ds(row0, ROWS_PER_TILE)
  pltpu.sync_copy(acc_s.at[sl], out_hbm.at[c, sl])


_BLK = 2048
_NBLK = R // _BLK


def _colsum(ref):
  s = jnp.sum(ref[...], axis=0, keepdims=True)
  return lax.transpose(s, (1, 0))


def _layer1_body(h_ref, od_ref, mask_ref, w_ref, out_ref):
  od = _colsum(od_ref)
  ns = lax.rsqrt(jnp.maximum(od, 1.0)) * mask_ref[...]
  x = h_ref[...] * ns
  out_ref[...] = lax.dot_general(
      x, w_ref[...], (((1,), (0,)), ((), ())),
      preferred_element_type=jnp.float32)


def _tc_layer1(h_pad, od2d, mask, W1):
  return pl.pallas_call(
      _layer1_body,
      grid=(_NBLK,),
      in_specs=[
          pl.BlockSpec((_BLK, D), lambda i: (i, 0)),
          pl.BlockSpec((NW, _BLK), lambda i: (0, i)),
          pl.BlockSpec((_BLK, 1), lambda i: (i, 0)),
          pl.BlockSpec((D, D), lambda i: (0, 0)),
      ],
      out_specs=pl.BlockSpec((_BLK, D), lambda i: (i, 0)),
      out_shape=jax.ShapeDtypeStruct((R, D), jnp.float32),
  )(h_pad, od2d, mask, W1)


def _layer2_body(agg_ref, od_ref, id_ref, mask_ref, b_ref, w_ref, out_ref):
  a = agg_ref[0] + agg_ref[1]
  nd = lax.rsqrt(jnp.maximum(_colsum(id_ref), 1.0))
  y = jnp.maximum(a * nd + b_ref[...], 0.0)
  ns = lax.rsqrt(jnp.maximum(_colsum(od_ref), 1.0)) * mask_ref[...]
  out_ref[...] = lax.dot_general(
      y * ns, w_ref[...], (((1,), (0,)), ((), ())),
      preferred_element_type=jnp.float32)


def _tc_layer2(agg1, od2d, id2d, mask, b1, W2):
  return pl.pallas_call(
      _layer2_body,
      grid=(_NBLK,),
      in_specs=[
          pl.BlockSpec((NC, _BLK, D), lambda i: (0, i, 0)),
          pl.BlockSpec((NW, _BLK), lambda i: (0, i)),
          pl.BlockSpec((NW, _BLK), lambda i: (0, i)),
          pl.BlockSpec((_BLK, 1), lambda i: (i, 0)),
          pl.BlockSpec((1, D), lambda i: (0, 0)),
          pl.BlockSpec((D, D), lambda i: (0, 0)),
      ],
      out_specs=pl.BlockSpec((_BLK, D), lambda i: (i, 0)),
      out_shape=jax.ShapeDtypeStruct((R, D), jnp.float32),
  )(agg1, od2d, id2d, mask, b1, W2)


def _head_body(agg_ref, id_ref, mask_ref, b_ref, wfc_ref, bfc_ref, out_ref,
               acc_ref):
  i = pl.program_id(0)

  @pl.when(i == 0)
  def _():
    acc_ref[...] = jnp.zeros_like(acc_ref)

  a = agg_ref[0] + agg_ref[1]
  nd = lax.rsqrt(jnp.maximum(_colsum(id_ref), 1.0))
  y = jnp.maximum(a * nd + b_ref[...], 0.0) * mask_ref[...]
  acc_ref[...] += jnp.sum(y, axis=0, keepdims=True)

  @pl.when(i == _NBLK - 1)
  def _():
    pooled = acc_ref[...] * (1.0 / N)
    out_ref[...] = jnp.sum(pooled * wfc_ref[...], axis=1, keepdims=True) \
        + bfc_ref[...]


def _tc_head(agg2, id2d, mask, b2, wfcT, bfc):
  return pl.pallas_call(
      _head_body,
      grid=(_NBLK,),
      in_specs=[
          pl.BlockSpec((NC, _BLK, D), lambda i: (0, i, 0)),
          pl.BlockSpec((NW, _BLK), lambda i: (0, i)),
          pl.BlockSpec((_BLK, 1), lambda i: (i, 0)),
          pl.BlockSpec((1, D), lambda i: (0, 0)),
          pl.BlockSpec((1, D), lambda i: (0, 0)),
          pl.BlockSpec((1, 1), lambda i: (0, 0)),
      ],
      out_specs=pl.BlockSpec((1, 1), lambda i: (0, 0)),
      out_shape=jax.ShapeDtypeStruct((1, 1), jnp.float32),
      scratch_shapes=[pltpu.VMEM((1, D), jnp.float32)],
  )(agg2, id2d, mask, b2, wfcT, bfc)


def kernel(h, edge_index, W1, b1, W2, b2, Wfc, bfc):
  ei = edge_index.astype(jnp.int32)
  pad = jnp.full((E_PAD - ei.shape[1],), N, dtype=jnp.int32)
  src = jnp.concatenate([ei[0], pad]).reshape(TOT_CHUNK, CH)
  dst = jnp.concatenate([ei[1], pad]).reshape(TOT_CHUNK, CH)
  h_pad = jnp.pad(h, ((0, R - N), (0, 0)))
  mask = jnp.pad(jnp.ones((N, 1), jnp.float32), ((0, R - N), (0, 0)))
  zrow = jnp.zeros((ROWS_PER_TILE, D), jnp.float32)

  od1, id1 = _deg_kernel(src, dst)
  od2d = od1.reshape(NW, R)
  id2d = id1.reshape(NW, R)

  xw1 = _tc_layer1(h_pad, od2d, mask, W1)
  agg1 = _spmm_kernel(xw1, src, dst, zrow)
  xw2 = _tc_layer2(agg1, od2d, id2d, mask, b1.reshape(1, D), W2)
  agg2 = _spmm_kernel(xw2, src, dst, zrow)
  out = _tc_head(agg2, id2d, mask, b2.reshape(1, D), Wfc.reshape(1, D),
                 bfc.reshape(1, 1))
  return out.reshape(1)


# speedup vs baseline: 1.1201x; 1.0107x over previous
import functools

import jax
import jax.numpy as jnp
from jax import lax
from jax.experimental import pallas as pl
from jax.experimental.pallas import tpu as pltpu
from jax.experimental.pallas import tpu_sc as plsc

N = 10000
D = 128
R = 10240
NC = 2
NS = 16
NW = NC * NS
CH = 128
NCHUNK = 80
E_PAD = NW * NCHUNK * CH
TOT_CHUNK = E_PAD // CH
CPW0 = 152
CPW1 = 2 * NCHUNK - CPW0
ROWS_PER_TILE = R // NS

_MESH = plsc.VectorSubcoreMesh(
    core_axis_name="c", subcore_axis_name="s", num_cores=NC, num_subcores=NS)


_PH = 32


def _chunk_start(c, s):
  return jnp.where(c == 0, s * CPW0, NS * CPW0 + s * CPW1)


def _phase_sizes(cnt):
  sizes = [_PH] * (cnt // _PH)
  if cnt % _PH:
    sizes.append(cnt % _PH)
  return sizes


@functools.partial(
    pl.kernel,
    out_type=(
        jax.ShapeDtypeStruct((NW * R,), jnp.float32),
        jax.ShapeDtypeStruct((NW * R,), jnp.float32),
    ),
    mesh=_MESH,
    scratch_types=[
        pltpu.VMEM((_PH, CH), jnp.int32),
        pltpu.VMEM((_PH, CH), jnp.int32),
        pltpu.VMEM((R,), jnp.float32),
        pltpu.VMEM((R,), jnp.float32),
    ],
    compiler_params=pltpu.CompilerParams(needs_layout_passes=False),
)
def _deg_kernel(src_hbm, dst_hbm, od_out, id_out,
                src_v, dst_v, od_h, id_h):
  c = lax.axis_index("c")
  s = lax.axis_index("s")
  w = c * NS + s
  start = _chunk_start(c, s)

  zero = jnp.zeros((16,), jnp.float32)

  def zbody(i, carry):
    od_h[pl.ds(i * 16, 16)] = zero
    id_h[pl.ds(i * 16, 16)] = zero
    return carry

  lax.fori_loop(0, R // 16, zbody, 0)

  ones = jnp.ones((16,), jnp.float32)

  def hist_phases(cnt):
    done = 0
    for sz in _phase_sizes(cnt):
      base = start + done
      pltpu.sync_copy(src_hbm.at[pl.ds(base, sz)], src_v.at[pl.ds(0, sz)])
      pltpu.sync_copy(dst_hbm.at[pl.ds(base, sz)], dst_v.at[pl.ds(0, sz)])

      def ebody(j, carry):
        for k in range(CH // 16):
          sidx = src_v[j, pl.ds(k * 16, 16)]
          plsc.addupdate_scatter(od_h, [sidx], ones)
          didx = dst_v[j, pl.ds(k * 16, 16)]
          plsc.addupdate_scatter(id_h, [didx], ones)
        return carry

      lax.fori_loop(0, sz, ebody, 0)
      done += sz

  @pl.when(c == 0)
  def _():
    hist_phases(CPW0)

  @pl.when(c == 1)
  def _():
    hist_phases(CPW1)

  pltpu.sync_copy(od_h, od_out.at[pl.ds(w * R, R)])
  pltpu.sync_copy(id_h, id_out.at[pl.ds(w * R, R)])


@functools.partial(
    pl.kernel,
    out_type=jax.ShapeDtypeStruct((NC, R, D), jnp.float32),
    mesh=_MESH,
    scratch_types=[
        pltpu.VMEM((_PH, CH), jnp.int32),
        pltpu.VMEM((_PH, CH), jnp.int32),
        pltpu.VMEM((CH, D), jnp.float32),
        pltpu.VMEM((CH, D), jnp.float32),
        pltpu.VMEM_SHARED((R, D), jnp.float32),
        pltpu.SemaphoreType.DMA,
        pltpu.SemaphoreType.DMA,
    ],
    compiler_params=pltpu.CompilerParams(needs_layout_passes=False),
)
def _spmm_kernel(x_hbm, src_hbm, dst_hbm,
                 out_hbm,
                 src_v, dst_v, rows0_v, rows1_v, acc_s, sem0, sem1):
  c = lax.axis_index("c")
  s = lax.axis_index("s")
  row0 = s * ROWS_PER_TILE
  start = _chunk_start(c, s)

  zero16 = jnp.zeros((16,), jnp.float32)

  def zb(j, carry):
    for k in range(D // 16):
      rows0_v[j, pl.ds(k * 16, 16)] = zero16
    return carry

  lax.fori_loop(0, CH, zb, 0)
  for t in range(ROWS_PER_TILE // CH):
    pltpu.sync_copy(rows0_v, acc_s.at[pl.ds(row0 + t * CH, CH)])
  plsc.subcore_barrier()

  def edge_phases(cnt):
    done = 0
    for sz in _phase_sizes(cnt):
      base = start + done
      pltpu.sync_copy(src_hbm.at[pl.ds(base, sz)], src_v.at[pl.ds(0, sz)])
      pltpu.sync_copy(dst_hbm.at[pl.ds(base, sz)], dst_v.at[pl.ds(0, sz)])
      pltpu.async_copy(x_hbm.at[src_v.at[0]], rows0_v, sem0)
      pltpu.async_copy(x_hbm.at[src_v.at[1]], rows1_v, sem1)

      def body(i, carry):
        j0 = 2 * i
        pltpu.make_async_copy(x_hbm.at[src_v.at[j0]], rows0_v, sem0).wait()
        pltpu.sync_copy(rows0_v, acc_s.at[dst_v.at[j0]], add=True)

        @pl.when(j0 + 2 < sz)
        def _():
          pltpu.async_copy(x_hbm.at[src_v.at[j0 + 2]], rows0_v, sem0)

        pltpu.make_async_copy(x_hbm.at[src_v.at[j0 + 1]], rows1_v, sem1).wait()
        pltpu.sync_copy(rows1_v, acc_s.at[dst_v.at[j0 + 1]], add=True)

        @pl.when(j0 + 3 < sz)
        def _():
          pltpu.async_copy(x_hbm.at[src_v.at[j0 + 3]], rows1_v, sem1)

        return carry

      lax.fori_loop(0, sz // 2, body, 0)
      done += sz

  @pl.when(c == 0)
  def _():
    edge_phases(CPW0)

  @pl.when(c == 1)
  def _():
    edge_phases(CPW1)
  plsc.subcore_barrier()
  sl = pl.ds(row0, ROWS_PER_TILE)
  pltpu.sync_copy(acc_s.at[sl], out_hbm.at[c, sl])


_BLK = 2048
_NBLK = R // _BLK


def _colsum(ref):
  s = jnp.sum(ref[...], axis=0, keepdims=True)
  return lax.transpose(s, (1, 0))


def _layer1_body(h_ref, od_ref, mask_ref, w_ref, out_ref):
  od = _colsum(od_ref)
  ns = lax.rsqrt(jnp.maximum(od, 1.0)) * mask_ref[...]
  x = h_ref[...] * ns
  out_ref[...] = lax.dot_general(
      x, w_ref[...], (((1,), (0,)), ((), ())),
      preferred_element_type=jnp.float32)


def _tc_layer1(h_pad, od2d, mask, W1):
  return pl.pallas_call(
      _layer1_body,
      grid=(_NBLK,),
      in_specs=[
          pl.BlockSpec((_BLK, D), lambda i: (i, 0)),
          pl.BlockSpec((NW, _BLK), lambda i: (0, i)),
          pl.BlockSpec((_BLK, 1), lambda i: (i, 0)),
          pl.BlockSpec((D, D), lambda i: (0, 0)),
      ],
      out_specs=pl.BlockSpec((_BLK, D), lambda i: (i, 0)),
      out_shape=jax.ShapeDtypeStruct((R, D), jnp.float32),
  )(h_pad, od2d, mask, W1)


def _layer2_body(agg_ref, od_ref, id_ref, mask_ref, b_ref, w_ref, out_ref):
  a = agg_ref[0] + agg_ref[1]
  nd = lax.rsqrt(jnp.maximum(_colsum(id_ref), 1.0))
  y = jnp.maximum(a * nd + b_ref[...], 0.0)
  ns = lax.rsqrt(jnp.maximum(_colsum(od_ref), 1.0)) * mask_ref[...]
  out_ref[...] = lax.dot_general(
      y * ns, w_ref[...], (((1,), (0,)), ((), ())),
      preferred_element_type=jnp.float32)


def _tc_layer2(agg1, od2d, id2d, mask, b1, W2):
  return pl.pallas_call(
      _layer2_body,
      grid=(_NBLK,),
      in_specs=[
          pl.BlockSpec((NC, _BLK, D), lambda i: (0, i, 0)),
          pl.BlockSpec((NW, _BLK), lambda i: (0, i)),
          pl.BlockSpec((NW, _BLK), lambda i: (0, i)),
          pl.BlockSpec((_BLK, 1), lambda i: (i, 0)),
          pl.BlockSpec((1, D), lambda i: (0, 0)),
          pl.BlockSpec((D, D), lambda i: (0, 0)),
      ],
      out_specs=pl.BlockSpec((_BLK, D), lambda i: (i, 0)),
      out_shape=jax.ShapeDtypeStruct((R, D), jnp.float32),
  )(agg1, od2d, id2d, mask, b1, W2)


def _head_body(agg_ref, id_ref, mask_ref, b_ref, wfc_ref, bfc_ref, out_ref,
               acc_ref):
  i = pl.program_id(0)

  @pl.when(i == 0)
  def _():
    acc_ref[...] = jnp.zeros_like(acc_ref)

  a = agg_ref[0] + agg_ref[1]
  nd = lax.rsqrt(jnp.maximum(_colsum(id_ref), 1.0))
  y = jnp.maximum(a * nd + b_ref[...], 0.0) * mask_ref[...]
  acc_ref[...] += jnp.sum(y, axis=0, keepdims=True)

  @pl.when(i == _NBLK - 1)
  def _():
    pooled = acc_ref[...] * (1.0 / N)
    out_ref[...] = jnp.sum(pooled * wfc_ref[...], axis=1, keepdims=True) \
        + bfc_ref[...]


def _tc_head(agg2, id2d, mask, b2, wfcT, bfc):
  return pl.pallas_call(
      _head_body,
      grid=(_NBLK,),
      in_specs=[
          pl.BlockSpec((NC, _BLK, D), lambda i: (0, i, 0)),
          pl.BlockSpec((NW, _BLK), lambda i: (0, i)),
          pl.BlockSpec((_BLK, 1), lambda i: (i, 0)),
          pl.BlockSpec((1, D), lambda i: (0, 0)),
          pl.BlockSpec((1, D), lambda i: (0, 0)),
          pl.BlockSpec((1, 1), lambda i: (0, 0)),
      ],
      out_specs=pl.BlockSpec((1, 1), lambda i: (0, 0)),
      out_shape=jax.ShapeDtypeStruct((1, 1), jnp.float32),
      scratch_shapes=[pltpu.VMEM((1, D), jnp.float32)],
  )(agg2, id2d, mask, b2, wfcT, bfc)


def kernel(h, edge_index, W1, b1, W2, b2, Wfc, bfc):
  ei = edge_index.astype(jnp.int32)
  pad = jnp.full((E_PAD - ei.shape[1],), N, dtype=jnp.int32)
  src = jnp.concatenate([ei[0], pad]).reshape(TOT_CHUNK, CH)
  dst = jnp.concatenate([ei[1], pad]).reshape(TOT_CHUNK, CH)
  h_pad = jnp.pad(h, ((0, R - N), (0, 0)))
  mask = jnp.pad(jnp.ones((N, 1), jnp.float32), ((0, R - N), (0, 0)))

  od1, id1 = _deg_kernel(src, dst)
  od2d = od1.reshape(NW, R)
  id2d = id1.reshape(NW, R)

  xw1 = _tc_layer1(h_pad, od2d, mask, W1)
  agg1 = _spmm_kernel(xw1, src, dst)
  xw2 = _tc_layer2(agg1, od2d, id2d, mask, b1.reshape(1, D), W2)
  agg2 = _spmm_kernel(xw2, src, dst)
  out = _tc_head(agg2, id2d, mask, b2.reshape(1, D), Wfc.reshape(1, D),
                 bfc.reshape(1, 1))
  return out.reshape(1)
